# baseline (speedup 1.0000x reference)
import functools

import jax
import jax.numpy as jnp
from jax import lax
from jax.experimental import pallas as pl
from jax.experimental.pallas import tpu as pltpu
from jax.experimental.pallas import tpu_sc as plsc

CHUNKS = (32, 64, 80, 80, 80, 80, 64, 32)


def _gather_call(B, V, D, dtype):
    info = plsc.get_sparse_core_info()
    NC, NS = info.num_cores, info.num_subcores
    NW = NC * NS
    b_per_w = B // NW
    assert sum(CHUNKS) == b_per_w
    n_chunk = len(CHUNKS)
    offs = [sum(CHUNKS[:j]) for j in range(n_chunk)]
    stage_main = 64
    stage_tail = V - 15 * stage_main

    mesh = plsc.VectorSubcoreMesh(core_axis_name="c", subcore_axis_name="s")

    @functools.partial(
        pl.kernel,
        mesh=mesh,
        out_type=jax.ShapeDtypeStruct((B, D), dtype),
        scratch_types=(
            [
                pltpu.VMEM_SHARED((V, D), dtype),
                pltpu.VMEM((b_per_w,), jnp.int32),
            ]
            + [pltpu.VMEM((c, D), dtype) for c in CHUNKS]
            + [pltpu.SemaphoreType.DMA for _ in range(2 * n_chunk)]
        ),
    )
    def k(t_hbm, pe_hbm, out_hbm, table_sh, idx_v, *bufs_and_sems):
        bufs = bufs_and_sems[:n_chunk]
        gsems = bufs_and_sems[n_chunk : 2 * n_chunk]
        osems = bufs_and_sems[2 * n_chunk :]
        c = lax.axis_index("c")
        s = lax.axis_index("s")
        wid = s * NC + c
        base = wid * b_per_w

        pltpu.sync_copy(t_hbm.at[pl.ds(base, b_per_w)], idx_v)
        g0 = pltpu.async_copy(
            pe_hbm.at[idx_v.at[pl.ds(0, CHUNKS[0])]], bufs[0], gsems[0]
        )

        @pl.when(s < 15)
        def _():
            pltpu.sync_copy(
                pe_hbm.at[pl.ds(s * stage_main, stage_main)],
                table_sh.at[pl.ds(s * stage_main, stage_main)],
            )

        @pl.when(s == 15)
        def _():
            pltpu.sync_copy(
                pe_hbm.at[pl.ds(15 * stage_main, stage_tail)],
                table_sh.at[pl.ds(15 * stage_main, stage_tail)],
            )

        plsc.subcore_barrier()

        gathers = [g0] + [
            pltpu.async_copy(
                table_sh.at[idx_v.at[pl.ds(offs[j], CHUNKS[j])]], bufs[j], gsems[j]
            )
            for j in range(1, n_chunk)
        ]
        outs = []
        for j in range(n_chunk):
            gathers[j].wait()
            outs.append(
                pltpu.async_copy(
                    bufs[j], out_hbm.at[pl.ds(base + offs[j], CHUNKS[j])], osems[j]
                )
            )
        for o in outs:
            o.wait()

    return k


def kernel(t, pe):
    t = t.astype(jnp.int32)
    if t.ndim > 1:
        t = jnp.squeeze(t, axis=-1)
    B = t.shape[0]
    V, D = pe.shape
    return _gather_call(B, V, D, pe.dtype)(t, pe)


# speedup vs baseline: 1.0048x; 1.0048x over previous
import functools

import jax
import jax.numpy as jnp
from jax import lax
from jax.experimental import pallas as pl
from jax.experimental.pallas import tpu as pltpu
from jax.experimental.pallas import tpu_sc as plsc

CHUNKS = (64, 64, 64, 64, 64, 64, 64, 64)


def _gather_call(B, V, D, dtype):
    info = plsc.get_sparse_core_info()
    NC, NS = info.num_cores, info.num_subcores
    NW = NC * NS
    b_per_w = B // NW
    assert sum(CHUNKS) == b_per_w
    n_chunk = len(CHUNKS)
    offs = [sum(CHUNKS[:j]) for j in range(n_chunk)]
    stage_main = 64
    stage_tail = V - 15 * stage_main

    mesh = plsc.VectorSubcoreMesh(core_axis_name="c", subcore_axis_name="s")

    @functools.partial(
        pl.kernel,
        mesh=mesh,
        out_type=jax.ShapeDtypeStruct((B, D), dtype),
        scratch_types=(
            [
                pltpu.VMEM_SHARED((V, D), dtype),
                pltpu.VMEM((b_per_w,), jnp.int32),
            ]
            + [pltpu.VMEM((c, D), dtype) for c in CHUNKS]
            + [pltpu.SemaphoreType.DMA for _ in range(2 * n_chunk)]
        ),
    )
    def k(t_hbm, pe_hbm, out_hbm, table_sh, idx_v, *bufs_and_sems):
        bufs = bufs_and_sems[:n_chunk]
        gsems = bufs_and_sems[n_chunk : 2 * n_chunk]
        osems = bufs_and_sems[2 * n_chunk :]
        c = lax.axis_index("c")
        s = lax.axis_index("s")
        wid = s * NC + c
        base = wid * b_per_w

        pltpu.sync_copy(t_hbm.at[pl.ds(base, b_per_w)], idx_v)
        g0 = pltpu.async_copy(
            pe_hbm.at[idx_v.at[pl.ds(0, CHUNKS[0])]], bufs[0], gsems[0]
        )

        @pl.when(s < 15)
        def _():
            pltpu.sync_copy(
                pe_hbm.at[pl.ds(s * stage_main, stage_main)],
                table_sh.at[pl.ds(s * stage_main, stage_main)],
            )

        @pl.when(s == 15)
        def _():
            pltpu.sync_copy(
                pe_hbm.at[pl.ds(15 * stage_main, stage_tail)],
                table_sh.at[pl.ds(15 * stage_main, stage_tail)],
            )

        plsc.subcore_barrier()

        gathers = [g0] + [
            pltpu.async_copy(
                table_sh.at[idx_v.at[pl.ds(offs[j], CHUNKS[j])]], bufs[j], gsems[j]
            )
            for j in range(1, n_chunk)
        ]
        outs = []
        for j in range(n_chunk):
            gathers[j].wait()
            outs.append(
                pltpu.async_copy(
                    bufs[j], out_hbm.at[pl.ds(base + offs[j], CHUNKS[j])], osems[j]
                )
            )
        for o in outs:
            o.wait()

    return k


def kernel(t, pe):
    t = t.astype(jnp.int32)
    if t.ndim > 1:
        t = jnp.squeeze(t, axis=-1)
    B = t.shape[0]
    V, D = pe.shape
    return _gather_call(B, V, D, pe.dtype)(t, pe)
